# baseline (speedup 1.0000x reference)
import functools

import numpy as np

import jax
import jax.numpy as jnp
from jax import lax
from jax.experimental import pallas as pl
from jax.experimental.pallas import tpu as pltpu

LW = 128


def _dot(a, w):
    return lax.dot_general(a, w, (((a.ndim - 1,), (0,)), ((), ())),
                           preferred_element_type=jnp.float32)


def _dot_t(a, w):
    return lax.dot_general(a, w, (((a.ndim - 1,), (w.ndim - 1,)), ((), ())),
                           preferred_element_type=jnp.float32)


def _dot_rec(a, w):
    return lax.dot_general(a, w, (((a.ndim - 1,), (0,)), ((), ())),
                           preferred_element_type=jnp.float32)


def _dot_pre(a, w):
    return lax.dot_general(a, w, (((a.ndim - 1,), (0,)), ((), ())),
                           preferred_element_type=jnp.float32)


def _dot_perm(a, w):
    return lax.dot_general(a, w, (((a.ndim - 1,), (0,)), ((), ())),
                           preferred_element_type=jnp.float32)


def _dot_q(a, w):
    return lax.dot_general(a, w, (((a.ndim - 1,), (0,)), ((), ())),
                           preferred_element_type=jnp.float32)


def _dot_epi(a, w):
    return lax.dot_general(a, w, (((a.ndim - 1,), (0,)), ((), ())),
                           preferred_element_type=jnp.float32)


def _branch(x_ref, pars, cstk_ref, msum_ref, scr, *, R, C, L, N, H, B4):
    g_ref, be_ref, wih_ref, bi_ref, whh_ref, wl_ref, lb_ref = pars
    pre2, hid_sm = scr
    W = B4 * C
    packed = 8 * H <= LW
    X = x_ref[0:R, 0:W]

    inv_n = 1.0 / (R * C)
    msum = msum_ref[...]
    s1 = jnp.sum(X, axis=0, keepdims=True)
    mu = _dot(s1, msum) * inv_n
    d = X - mu
    s2 = jnp.sum(d * d, axis=0, keepdims=True)
    var = _dot(s2, msum) * inv_n
    rs = lax.rsqrt(var + 1e-5)
    gam, bet = g_ref[...], be_ref[...]
    grep = jnp.concatenate([gam] * B4, axis=1)
    brep = jnp.concatenate([bet] * B4, axis=1)
    xn = d * (rs * grep) + brep

    cstk = cstk_ref[...]

    if packed:
        G4 = 4 * H
        zc = jnp.zeros((C, LW - G4), jnp.float32)
        dirs = (
            (jnp.concatenate([wih_ref[:, 0:G4], zc], axis=1), False),
            (jnp.concatenate([zc, wih_ref[:, LW:LW + G4]], axis=1), True),
        )
        zb = jnp.zeros((1, LW - G4), jnp.float32)
        bias_t = (jnp.concatenate([bi_ref[:, 0:G4], zb], axis=1)
                  + jnp.concatenate([zb, bi_ref[:, LW:LW + G4]], axis=1))
        GW = LW
    else:
        dirs = (
            (wih_ref[:, 0:LW], False),
            (wih_ref[:, LW:2 * LW], True),
        )
        bias_t = jnp.concatenate([bi_ref[:, 0:LW], bi_ref[:, LW:2 * LW]],
                                 axis=1)
        GW = 2 * LW

    PDs = []
    for wihD, _rev in dirs:
        wih_ext = jnp.concatenate(
            [lax.dot_general(cstk[b], wihD, (((0,), (0,)), ((), ())),
                             preferred_element_type=jnp.float32)
             for b in range(B4)], axis=1)
        PDs.append(_dot_pre(xn, wih_ext))
    for b in range(B4):
        pf = PDs[0][:, LW * b:LW * (b + 1)]
        pb = PDs[1][:, LW * b:LW * (b + 1)]
        if packed:
            for l in range(L):
                pre2[b, l * N:(l + 1) * N, 0:LW] = (
                    pf[l * N:(l + 1) * N, :]
                    + pb[(L - 1 - l) * N:(L - l) * N, :] + bias_t)
        else:
            pre2[b, 0:R, 0:LW] = pf + bias_t[:, 0:LW]
            for l in range(L):
                pre2[b, l * N:(l + 1) * N, LW:2 * LW] = (
                    pb[(L - 1 - l) * N:(L - l) * N, :] + bias_t[:, LW:2 * LW])

    def cell(gts, c):
        sg = jax.nn.sigmoid(gts)
        tg = jnp.tanh(gts)
        f_al = pltpu.roll(sg, LW - H, axis=2)
        g_al = pltpu.roll(tg, LW - 2 * H, axis=2)
        o_al = pltpu.roll(sg, LW - 3 * H, axis=2)
        c = f_al * c + sg * g_al
        return o_al * jnp.tanh(c), c

    z = jnp.zeros((B4, N, LW), jnp.float32)

    if packed:
        HB = LW // 2
        zq = jnp.zeros((HB, HB), jnp.float32)
        whh_pk = jnp.concatenate(
            [jnp.concatenate([whh_ref[0:HB, 0:HB], zq], axis=1),
             jnp.concatenate([zq, whh_ref[LW:LW + HB, LW:LW + HB]], axis=1)],
            axis=0)

        def step(i, carry):
            h, c = carry
            rf = i * N
            if not isinstance(i, int):
                rf = pl.multiple_of(rf, 8)
            g = pre2[:, pl.ds(rf, N), 0:LW] + _dot_rec(h, whh_pk)
            h, c = cell(g, c)
            hid_sm[:, pl.ds(rf, N), 0:LW] = h
            return h, c

        carry = (z, z)
    else:
        whhF = whh_ref[0:LW, 0:LW]
        whhB = whh_ref[LW:2 * LW, LW:2 * LW]

        def step(i, carry):
            hf, cf, hb, cb = carry
            rf = i * N
            if not isinstance(i, int):
                rf = pl.multiple_of(rf, 8)
            rd = pre2[:, pl.ds(rf, N), :]
            gf = rd[:, :, 0:LW] + _dot_rec(hf, whhF)
            gb = rd[:, :, LW:2 * LW] + _dot_rec(hb, whhB)
            hf, cf = cell(gf, cf)
            hb, cb = cell(gb, cb)
            hid_sm[:, pl.ds(rf, N), :] = jnp.concatenate([hf, hb], axis=2)
            return hf, cf, hb, cb

        carry = (z, z, z, z)

    if N % 8 == 0:
        carry = lax.fori_loop(0, L, step, carry, unroll=2)
    else:
        for i in range(L):
            carry = step(i, carry)

    wlF = wl_ref[0:LW, :]
    if packed:
        wlB = jnp.concatenate(
            [jnp.zeros((LW // 2, C), jnp.float32),
             wl_ref[LW:LW + LW // 2, :]], axis=0)
        hsF = hid_sm[:, 0:R, 0:LW]
        hsB = hsF
    else:
        wlB = wl_ref[LW:2 * LW, :]
        hsF = hid_sm[:, 0:R, 0:LW]
        hsB = hid_sm[:, 0:R, LW:2 * LW]
    yf = _dot_epi(hsF, wlF)
    yb = _dot_epi(hsB, wlB)
    ybr = jnp.concatenate(
        [yb[:, (L - 1 - l) * N:(L - l) * N, :] for l in range(L)], axis=1)
    y3 = yf + ybr
    lrep = jnp.concatenate([lb_ref[...]] * B4, axis=1)
    acc = _dot(y3[0], cstk[0])
    for b in range(1, B4):
        acc = acc + _dot(y3[b], cstk[b])
    return acc + lrep + X


def _layer(a_ref, b_ref, perm_ref, q1_ref, q2_ref, p0, p1,
           cstk_ref, msum_ref, er_ref, ei_ref, scr,
           *, R_in, C_in, R_out, F, H, inverse, B4):
    T_in = R_in // F
    W = B4 * C_in
    y0 = _branch(a_ref, p0, cstk_ref, msum_ref, scr,
                 R=R_in, C=C_in, L=F, N=T_in, H=H, B4=B4)
    b_ref[0:R_in, 0:W] = _dot_perm(perm_ref[...], y0)
    y1 = _branch(b_ref, p1, cstk_ref, msum_ref, scr,
                 R=R_in, C=C_in, L=T_in, N=F, H=H, B4=B4)
    if not inverse:
        z1 = _dot_q(q1_ref[...], y1)
        z2 = _dot_q(q2_ref[...], y1)
        return _dot(z1, er_ref[...]) + _dot(z2, ei_ref[...])
    yr = _dot(y1, er_ref[...])
    yi = _dot(y1, ei_ref[...])
    return _dot_q(q1_ref[...], yr) + _dot_q(q2_ref[...], yi)


def _net_kernel(*args, B4, cfgs):
    x_ref = args[0]
    params = args[1:1 + 17 * 4]
    (cf_ref, ci_ref, mf_ref, mi_ref, er_ref, ei_ref,
     ert_ref, eit_ref, cft_ref) = args[69:78]
    o_ref = args[78]
    a_s, b_s, pre2, hid_sm = args[79:]
    scr = (pre2, hid_sm)

    xin = x_ref[...]
    cf = cf_ref[...]
    R0, C0 = xin.shape[1], xin.shape[2]
    acc = _dot(xin[0], cf[0])
    for b in range(1, B4):
        acc = acc + _dot(xin[b], cf[b])
    a_s[0:R0, 0:B4 * C0] = acc

    for li, cfg in enumerate(cfgs):
        p = params[li * 17:(li + 1) * 17]
        fwd = not cfg['inverse']
        out_val = _layer(
            a_s, b_s, p[0], p[1], p[2], p[3:10], p[10:17],
            cf_ref if fwd else ci_ref, mf_ref if fwd else mi_ref,
            ert_ref if fwd else er_ref, eit_ref if fwd else ei_ref,
            scr, B4=B4, **cfg)
        if li == 3:
            cft = cft_ref[...]
            for b in range(B4):
                o_ref[b, 0:cfg['R_out'], 0:C0] = _dot(out_val, cft[b])
        else:
            Wn = out_val.shape[1]
            a_s[0:cfg['R_out'], 0:Wn] = out_val


def kernel(x, l0_perm, l0_q1, l0_q2, l0_e1, l0_e2, l0_g0, l0_be0, l0_wih0, l0_bi0, l0_whh0, l0_wl0, l0_lb0, l0_g1, l0_be1, l0_wih1, l0_bi1, l0_whh1, l0_wl1, l0_lb1, l1_perm, l1_q1, l1_q2, l1_e1, l1_e2, l1_g0, l1_be0, l1_wih0, l1_bi0, l1_whh0, l1_wl0, l1_lb0, l1_g1, l1_be1, l1_wih1, l1_bi1, l1_whh1, l1_wl1, l1_lb1, l2_perm, l2_q1, l2_q2, l2_e1, l2_e2, l2_g0, l2_be0, l2_wih0, l2_bi0, l2_whh0, l2_wl0, l2_lb0, l2_g1, l2_be1, l2_wih1, l2_bi1, l2_whh1, l2_wl1, l2_lb1, l3_perm, l3_q1, l3_q2, l3_e1, l3_e2, l3_g0, l3_be0, l3_wih0, l3_bi0, l3_whh0, l3_wl0, l3_lb0, l3_g1, l3_be1, l3_wih1, l3_bi1, l3_whh1, l3_wl1, l3_lb1):
    B, C, F, T = x.shape
    a = jnp.transpose(x, (0, 2, 3, 1)).reshape(B, F * T, C)

    per_layer = [
        [l0_perm, l0_q1, l0_q2, l0_g0, l0_be0, l0_wih0, l0_bi0, l0_whh0,
         l0_wl0, l0_lb0, l0_g1, l0_be1, l0_wih1, l0_bi1, l0_whh1, l0_wl1, l0_lb1],
        [l1_perm, l1_q1, l1_q2, l1_g0, l1_be0, l1_wih0, l1_bi0, l1_whh0,
         l1_wl0, l1_lb0, l1_g1, l1_be1, l1_wih1, l1_bi1, l1_whh1, l1_wl1, l1_lb1],
        [l2_perm, l2_q1, l2_q2, l2_g0, l2_be0, l2_wih0, l2_bi0, l2_whh0,
         l2_wl0, l2_lb0, l2_g1, l2_be1, l2_wih1, l2_bi1, l2_whh1, l2_wl1, l2_lb1],
        [l3_perm, l3_q1, l3_q2, l3_g0, l3_be0, l3_wih0, l3_bi0, l3_whh0,
         l3_wl0, l3_lb0, l3_g1, l3_be1, l3_wih1, l3_bi1, l3_whh1, l3_wl1, l3_lb1],
    ]
    consts = [arr for lp in per_layer for arr in lp]

    B4 = B // 2
    K = T // 2 + 1
    Rf, Ri = F * T, F * K
    C2 = 2 * C
    Wf, Wi = B4 * C, B4 * C2

    cf = np.zeros((B4, C, Wf), np.float32)
    for b in range(B4):
        cf[b, np.arange(C), b * C + np.arange(C)] = 1.0
    ci = np.zeros((B4, C2, Wi), np.float32)
    for b in range(B4):
        ci[b, np.arange(C2), b * C2 + np.arange(C2)] = 1.0
    mf = (np.add.outer(np.arange(Wf) // C, -(np.arange(Wf) // C)) == 0
          ).astype(np.float32)
    mi = (np.add.outer(np.arange(Wi) // C2, -(np.arange(Wi) // C2)) == 0
          ).astype(np.float32)
    er = np.zeros((Wi, Wf), np.float32)
    ei = np.zeros((Wi, Wf), np.float32)
    for j in range(Wf):
        er[C2 * (j // C) + (j % C), j] = 1.0
        ei[C2 * (j // C) + C + (j % C), j] = 1.0
    layout = [jnp.asarray(v) for v in
              (cf, ci, mf, mi, er, ei,
               er.T.copy(), ei.T.copy(), cf.transpose(0, 2, 1).copy())]

    cfgs = (
        dict(R_in=Rf, C_in=C, R_out=Ri, F=F, H=2 * C, inverse=False),
        dict(R_in=Ri, C_in=C2, R_out=Rf, F=F, H=4 * C, inverse=True),
        dict(R_in=Rf, C_in=C, R_out=Ri, F=F, H=2 * C, inverse=False),
        dict(R_in=Ri, C_in=C2, R_out=Rf, F=F, H=4 * C, inverse=True),
    )

    body = functools.partial(_net_kernel, B4=B4, cfgs=cfgs)
    out = pl.pallas_call(
        body,
        grid=(2,),
        in_specs=([pl.BlockSpec((B4, Rf, C), lambda i: (i, 0, 0))]
                  + [pl.BlockSpec(c.shape, functools.partial(
                        lambda n, i: (0,) * n, c.ndim))
                     for c in consts + layout]),
        out_specs=pl.BlockSpec((B4, Rf, C), lambda i: (i, 0, 0)),
        out_shape=jax.ShapeDtypeStruct((B, Rf, C), jnp.float32),
        scratch_shapes=[
            pltpu.VMEM((Rf, Wi), jnp.float32),
            pltpu.VMEM((Rf, Wi), jnp.float32),
            pltpu.VMEM((B4, Rf, 2 * LW), jnp.float32),
            pltpu.VMEM((B4, Rf, 2 * LW), jnp.float32),
        ],
        compiler_params=pltpu.CompilerParams(
            dimension_semantics=("parallel",)),
    )(a, *consts, *layout)
    return jnp.transpose(out.reshape(B, F, T, C), (0, 3, 1, 2))


# speedup vs baseline: 1.0593x; 1.0593x over previous
import functools

import numpy as np

import jax
import jax.numpy as jnp
from jax import lax
from jax.experimental import pallas as pl
from jax.experimental.pallas import tpu as pltpu

LW = 128


def _dot(a, w):
    return lax.dot_general(a, w, (((a.ndim - 1,), (0,)), ((), ())),
                           preferred_element_type=jnp.float32)


def _dot_t(a, w):
    return lax.dot_general(a, w, (((a.ndim - 1,), (w.ndim - 1,)), ((), ())),
                           preferred_element_type=jnp.float32)


def _dot_rec(a, w):
    return lax.dot_general(a, w, (((a.ndim - 1,), (0,)), ((), ())),
                           preferred_element_type=jnp.float32)


def _dot_pre(a, w):
    return lax.dot_general(a, w, (((a.ndim - 1,), (0,)), ((), ())),
                           preferred_element_type=jnp.float32)


def _dot_perm(a, w):
    return lax.dot_general(a, w, (((a.ndim - 1,), (0,)), ((), ())),
                           preferred_element_type=jnp.float32)


def _dot_q(a, w):
    return lax.dot_general(a, w, (((a.ndim - 1,), (0,)), ((), ())),
                           preferred_element_type=jnp.float32)


def _dot_epi(a, w):
    return lax.dot_general(a, w, (((a.ndim - 1,), (0,)), ((), ())),
                           preferred_element_type=jnp.float32)


def _branch(x_ref, pars, cstk_ref, msum_ref, scr, *, R, C, L, N, H, B4):
    g_ref, be_ref, wih_ref, bi_ref, whh_ref, wl_ref, lb_ref = pars
    pre2, hid_sm = scr
    W = B4 * C
    packed = 8 * H <= LW
    X = x_ref[0:R, 0:W]

    inv_n = 1.0 / (R * C)
    msum = msum_ref[...]
    s1 = jnp.sum(X, axis=0, keepdims=True)
    mu = _dot(s1, msum) * inv_n
    d = X - mu
    s2 = jnp.sum(d * d, axis=0, keepdims=True)
    var = _dot(s2, msum) * inv_n
    rs = lax.rsqrt(var + 1e-5)
    gam, bet = g_ref[...], be_ref[...]
    grep = jnp.concatenate([gam] * B4, axis=1)
    brep = jnp.concatenate([bet] * B4, axis=1)
    xn = d * (rs * grep) + brep

    cstk = cstk_ref[...]

    if packed:
        G4 = 4 * H
        zc = jnp.zeros((C, LW - G4), jnp.float32)
        zb = jnp.zeros((1, LW - G4), jnp.float32)
        dirs = (
            (0, jnp.concatenate([wih_ref[:, 0:G4], zc], axis=1),
             jnp.concatenate([bi_ref[:, 0:G4], zb], axis=1)),
            (LW, jnp.concatenate([zc, wih_ref[:, LW:LW + G4]], axis=1),
             jnp.concatenate([zb, bi_ref[:, LW:LW + G4]], axis=1)),
        )
    else:
        dirs = (
            (0, wih_ref[:, 0:LW], bi_ref[:, 0:LW]),
            (LW, wih_ref[:, LW:2 * LW], bi_ref[:, LW:2 * LW]),
        )

    for lane0, wihD, biD in dirs:
        wih_ext = jnp.concatenate(
            [lax.dot_general(cstk[b], wihD, (((0,), (0,)), ((), ())),
                             preferred_element_type=jnp.float32)
             for b in range(B4)], axis=1)
        PD = _dot_pre(xn, wih_ext)
        for b in range(B4):
            pre2[b, 0:R, lane0:lane0 + LW] = (
                PD[:, LW * b:LW * (b + 1)] + biD)

    lane = lax.broadcasted_iota(jnp.int32, (1, 1, LW), 2) % (4 * H)
    mhalf = jnp.where((lane >= 2 * H) & (lane < 3 * H), 1.0, 0.5)

    def cell(gts, c):
        u = jnp.tanh(gts * mhalf)
        f_al = pltpu.roll(u, LW - H, axis=2)
        g_al = pltpu.roll(u, LW - 2 * H, axis=2)
        o_al = pltpu.roll(u, LW - 3 * H, axis=2)
        c = (0.5 + 0.5 * f_al) * c + (0.5 + 0.5 * u) * g_al
        return (0.5 + 0.5 * o_al) * jnp.tanh(c), c

    z = jnp.zeros((B4, N, LW), jnp.float32)

    if packed:
        HB = LW // 2
        zq = jnp.zeros((HB, HB), jnp.float32)
        whh_pk = jnp.concatenate(
            [jnp.concatenate([whh_ref[0:HB, 0:HB], zq], axis=1),
             jnp.concatenate([zq, whh_ref[LW:LW + HB, LW:LW + HB]], axis=1)],
            axis=0)

        def step(i, carry):
            h, c = carry
            rf = i * N
            rb = (L - 1 - i) * N
            if not isinstance(i, int):
                rf = pl.multiple_of(rf, 8)
                rb = pl.multiple_of(rb, 8)
            g = (pre2[:, pl.ds(rf, N), 0:LW]
                 + pre2[:, pl.ds(rb, N), LW:2 * LW]
                 + _dot_rec(h, whh_pk))
            h, c = cell(g, c)
            for b in range(B4):
                hid_sm[pl.ds(rf, N), LW * b:LW * (b + 1)] = h[b]
                hid_sm[pl.ds(rb, N), LW * (B4 + b):LW * (B4 + b + 1)] = h[b]
            return h, c

        carry = (z, z)
    else:
        whhF = whh_ref[0:LW, 0:LW]
        whhB = whh_ref[LW:2 * LW, LW:2 * LW]

        def step(i, carry):
            hf, cf, hb, cb = carry
            rf = i * N
            rb = (L - 1 - i) * N
            if not isinstance(i, int):
                rf = pl.multiple_of(rf, 8)
                rb = pl.multiple_of(rb, 8)
            gf = pre2[:, pl.ds(rf, N), 0:LW] + _dot_rec(hf, whhF)
            gb = pre2[:, pl.ds(rb, N), LW:2 * LW] + _dot_rec(hb, whhB)
            hf, cf = cell(gf, cf)
            hb, cb = cell(gb, cb)
            for b in range(B4):
                hid_sm[pl.ds(rf, N), LW * b:LW * (b + 1)] = hf[b]
                hid_sm[pl.ds(rb, N), LW * (B4 + b):LW * (B4 + b + 1)] = hb[b]
            return hf, cf, hb, cb

        carry = (z, z, z, z)

    if N % 8 == 0:
        carry = lax.fori_loop(0, L, step, carry, unroll=4)
    else:
        for i in range(L):
            carry = step(i, carry)

    wlF = wl_ref[0:LW, :]
    if packed:
        wlB = jnp.concatenate(
            [jnp.zeros((LW // 2, C), jnp.float32),
             wl_ref[LW:LW + LW // 2, :]], axis=0)
    else:
        wlB = wl_ref[LW:2 * LW, :]
    wbigF = jnp.concatenate(
        [_dot(wlF, cstk[b]) for b in range(B4)], axis=0)
    wbigB = jnp.concatenate(
        [_dot(wlB, cstk[b]) for b in range(B4)], axis=0)
    lrep = jnp.concatenate([lb_ref[...]] * B4, axis=1)
    return (_dot_epi(hid_sm[0:R, 0:B4 * LW], wbigF)
            + _dot_epi(hid_sm[0:R, B4 * LW:2 * B4 * LW], wbigB)
            + lrep + X)


def _layer(a_ref, b_ref, perm_ref, q1_ref, q2_ref, p0, p1,
           cstk_ref, msum_ref, er_ref, ei_ref, scr,
           *, R_in, C_in, R_out, F, H, inverse, B4):
    T_in = R_in // F
    W = B4 * C_in
    y0 = _branch(a_ref, p0, cstk_ref, msum_ref, scr,
                 R=R_in, C=C_in, L=F, N=T_in, H=H, B4=B4)
    b_ref[0:R_in, 0:W] = _dot_perm(perm_ref[...], y0)
    y1 = _branch(b_ref, p1, cstk_ref, msum_ref, scr,
                 R=R_in, C=C_in, L=T_in, N=F, H=H, B4=B4)
    if not inverse:
        z1 = _dot_q(q1_ref[...], y1)
        z2 = _dot_q(q2_ref[...], y1)
        return _dot(z1, er_ref[...]) + _dot(z2, ei_ref[...])
    yr = _dot(y1, er_ref[...])
    yi = _dot(y1, ei_ref[...])
    return _dot_q(q1_ref[...], yr) + _dot_q(q2_ref[...], yi)


def _net_kernel(*args, B4, cfgs):
    x_ref = args[0]
    params = args[1:1 + 17 * 4]
    (cf_ref, ci_ref, mf_ref, mi_ref, er_ref, ei_ref,
     ert_ref, eit_ref, cft_ref) = args[69:78]
    o_ref = args[78]
    a_s, b_s, pre2, hid_sm = args[79:]
    scr = (pre2, hid_sm)

    xin = x_ref[...]
    cf = cf_ref[...]
    R0, C0 = xin.shape[1], xin.shape[2]
    acc = _dot(xin[0], cf[0])
    for b in range(1, B4):
        acc = acc + _dot(xin[b], cf[b])
    a_s[0:R0, 0:B4 * C0] = acc

    for li, cfg in enumerate(cfgs):
        p = params[li * 17:(li + 1) * 17]
        fwd = not cfg['inverse']
        out_val = _layer(
            a_s, b_s, p[0], p[1], p[2], p[3:10], p[10:17],
            cf_ref if fwd else ci_ref, mf_ref if fwd else mi_ref,
            ert_ref if fwd else er_ref, eit_ref if fwd else ei_ref,
            scr, B4=B4, **cfg)
        if li == 3:
            cft = cft_ref[...]
            for b in range(B4):
                o_ref[b, 0:cfg['R_out'], 0:C0] = _dot(out_val, cft[b])
        else:
            Wn = out_val.shape[1]
            a_s[0:cfg['R_out'], 0:Wn] = out_val


def kernel(x, l0_perm, l0_q1, l0_q2, l0_e1, l0_e2, l0_g0, l0_be0, l0_wih0, l0_bi0, l0_whh0, l0_wl0, l0_lb0, l0_g1, l0_be1, l0_wih1, l0_bi1, l0_whh1, l0_wl1, l0_lb1, l1_perm, l1_q1, l1_q2, l1_e1, l1_e2, l1_g0, l1_be0, l1_wih0, l1_bi0, l1_whh0, l1_wl0, l1_lb0, l1_g1, l1_be1, l1_wih1, l1_bi1, l1_whh1, l1_wl1, l1_lb1, l2_perm, l2_q1, l2_q2, l2_e1, l2_e2, l2_g0, l2_be0, l2_wih0, l2_bi0, l2_whh0, l2_wl0, l2_lb0, l2_g1, l2_be1, l2_wih1, l2_bi1, l2_whh1, l2_wl1, l2_lb1, l3_perm, l3_q1, l3_q2, l3_e1, l3_e2, l3_g0, l3_be0, l3_wih0, l3_bi0, l3_whh0, l3_wl0, l3_lb0, l3_g1, l3_be1, l3_wih1, l3_bi1, l3_whh1, l3_wl1, l3_lb1):
    B, C, F, T = x.shape
    a = jnp.transpose(x, (0, 2, 3, 1)).reshape(B, F * T, C)

    per_layer = [
        [l0_perm, l0_q1, l0_q2, l0_g0, l0_be0, l0_wih0, l0_bi0, l0_whh0,
         l0_wl0, l0_lb0, l0_g1, l0_be1, l0_wih1, l0_bi1, l0_whh1, l0_wl1, l0_lb1],
        [l1_perm, l1_q1, l1_q2, l1_g0, l1_be0, l1_wih0, l1_bi0, l1_whh0,
         l1_wl0, l1_lb0, l1_g1, l1_be1, l1_wih1, l1_bi1, l1_whh1, l1_wl1, l1_lb1],
        [l2_perm, l2_q1, l2_q2, l2_g0, l2_be0, l2_wih0, l2_bi0, l2_whh0,
         l2_wl0, l2_lb0, l2_g1, l2_be1, l2_wih1, l2_bi1, l2_whh1, l2_wl1, l2_lb1],
        [l3_perm, l3_q1, l3_q2, l3_g0, l3_be0, l3_wih0, l3_bi0, l3_whh0,
         l3_wl0, l3_lb0, l3_g1, l3_be1, l3_wih1, l3_bi1, l3_whh1, l3_wl1, l3_lb1],
    ]
    consts = [arr for lp in per_layer for arr in lp]

    B4 = B // 2
    K = T // 2 + 1
    Rf, Ri = F * T, F * K
    C2 = 2 * C
    Wf, Wi = B4 * C, B4 * C2

    cf = np.zeros((B4, C, Wf), np.float32)
    for b in range(B4):
        cf[b, np.arange(C), b * C + np.arange(C)] = 1.0
    ci = np.zeros((B4, C2, Wi), np.float32)
    for b in range(B4):
        ci[b, np.arange(C2), b * C2 + np.arange(C2)] = 1.0
    mf = (np.add.outer(np.arange(Wf) // C, -(np.arange(Wf) // C)) == 0
          ).astype(np.float32)
    mi = (np.add.outer(np.arange(Wi) // C2, -(np.arange(Wi) // C2)) == 0
          ).astype(np.float32)
    er = np.zeros((Wi, Wf), np.float32)
    ei = np.zeros((Wi, Wf), np.float32)
    for j in range(Wf):
        er[C2 * (j // C) + (j % C), j] = 1.0
        ei[C2 * (j // C) + C + (j % C), j] = 1.0
    layout = [jnp.asarray(v) for v in
              (cf, ci, mf, mi, er, ei,
               er.T.copy(), ei.T.copy(), cf.transpose(0, 2, 1).copy())]

    cfgs = (
        dict(R_in=Rf, C_in=C, R_out=Ri, F=F, H=2 * C, inverse=False),
        dict(R_in=Ri, C_in=C2, R_out=Rf, F=F, H=4 * C, inverse=True),
        dict(R_in=Rf, C_in=C, R_out=Ri, F=F, H=2 * C, inverse=False),
        dict(R_in=Ri, C_in=C2, R_out=Rf, F=F, H=4 * C, inverse=True),
    )

    body = functools.partial(_net_kernel, B4=B4, cfgs=cfgs)
    out = pl.pallas_call(
        body,
        grid=(2,),
        in_specs=([pl.BlockSpec((B4, Rf, C), lambda i: (i, 0, 0))]
                  + [pl.BlockSpec(c.shape, functools.partial(
                        lambda n, i: (0,) * n, c.ndim))
                     for c in consts + layout]),
        out_specs=pl.BlockSpec((B4, Rf, C), lambda i: (i, 0, 0)),
        out_shape=jax.ShapeDtypeStruct((B, Rf, C), jnp.float32),
        scratch_shapes=[
            pltpu.VMEM((Rf, Wi), jnp.float32),
            pltpu.VMEM((Rf, Wi), jnp.float32),
            pltpu.VMEM((B4, Rf, 2 * LW), jnp.float32),
            pltpu.VMEM((Rf, 2 * B4 * LW), jnp.float32),
        ],
        compiler_params=pltpu.CompilerParams(
            dimension_semantics=("parallel",)),
    )(a, *consts, *layout)
    return jnp.transpose(out.reshape(B, F, T, C), (0, 3, 1, 2))


# speedup vs baseline: 1.0765x; 1.0162x over previous
import functools

import numpy as np

import jax
import jax.numpy as jnp
from jax import lax
from jax.experimental import pallas as pl
from jax.experimental.pallas import tpu as pltpu

LW = 128


def _dot(a, w):
    return lax.dot_general(a, w, (((a.ndim - 1,), (0,)), ((), ())),
                           preferred_element_type=jnp.float32)


def _dot_t(a, w):
    return lax.dot_general(a, w, (((a.ndim - 1,), (w.ndim - 1,)), ((), ())),
                           preferred_element_type=jnp.float32)


def _dot_rec(a, w):
    return lax.dot_general(a, w, (((a.ndim - 1,), (0,)), ((), ())),
                           preferred_element_type=jnp.float32)


def _dot_pre(a, w):
    return lax.dot_general(a, w, (((a.ndim - 1,), (0,)), ((), ())),
                           preferred_element_type=jnp.float32)


def _dot_perm(a, w):
    return lax.dot_general(a, w, (((a.ndim - 1,), (0,)), ((), ())),
                           preferred_element_type=jnp.float32)


def _dot_q(a, w):
    return lax.dot_general(a, w, (((a.ndim - 1,), (0,)), ((), ())),
                           preferred_element_type=jnp.float32)


def _dot_epi(a, w):
    return lax.dot_general(a, w, (((a.ndim - 1,), (0,)), ((), ())),
                           preferred_element_type=jnp.float32)


def _branch(x_ref, pars, cstk_ref, msum_ref, scr, *, R, C, L, N, H, B4):
    g_ref, be_ref, wih_ref, bi_ref, whh_ref, wl_ref, lb_ref = pars
    pre2, hid_sm = scr
    W = B4 * C
    packed = 8 * H <= LW
    X = x_ref[0:R, 0:W]

    inv_n = 1.0 / (R * C)
    msum = msum_ref[...]
    s1 = jnp.sum(X, axis=0, keepdims=True)
    mu = _dot(s1, msum) * inv_n
    d = X - mu
    s2 = jnp.sum(d * d, axis=0, keepdims=True)
    var = _dot(s2, msum) * inv_n
    rs = lax.rsqrt(var + 1e-5)
    gam, bet = g_ref[...], be_ref[...]
    grep = jnp.concatenate([gam] * B4, axis=1)
    brep = jnp.concatenate([bet] * B4, axis=1)
    xn = d * (rs * grep) + brep

    cstk = cstk_ref[...]

    if packed:
        G4 = 4 * H
        zc = jnp.zeros((C, LW - G4), jnp.float32)
        zb = jnp.zeros((1, LW - G4), jnp.float32)
        dirs = (
            (0, jnp.concatenate([wih_ref[:, 0:G4], zc], axis=1),
             jnp.concatenate([bi_ref[:, 0:G4], zb], axis=1)),
            (LW, jnp.concatenate([zc, wih_ref[:, LW:LW + G4]], axis=1),
             jnp.concatenate([zb, bi_ref[:, LW:LW + G4]], axis=1)),
        )
    else:
        dirs = (
            (0, wih_ref[:, 0:LW], bi_ref[:, 0:LW]),
            (LW, wih_ref[:, LW:2 * LW], bi_ref[:, LW:2 * LW]),
        )

    for lane0, wihD, biD in dirs:
        wih_ext = jnp.concatenate(
            [lax.dot_general(cstk[b], wihD, (((0,), (0,)), ((), ())),
                             preferred_element_type=jnp.float32)
             for b in range(B4)], axis=1)
        PD = _dot_pre(xn, wih_ext)
        for b in range(B4):
            pre2[b, 0:R, lane0:lane0 + LW] = (
                PD[:, LW * b:LW * (b + 1)] + biD)

    lane = lax.broadcasted_iota(jnp.int32, (1, 1, LW), 2) % (4 * H)
    mhalf = jnp.where((lane >= 2 * H) & (lane < 3 * H), 1.0, 0.5)

    def cell(gts, c):
        u = jnp.tanh(gts * mhalf)
        f_al = pltpu.roll(u, LW - H, axis=2)
        g_al = pltpu.roll(u, LW - 2 * H, axis=2)
        o_al = pltpu.roll(u, LW - 3 * H, axis=2)
        c = (0.5 + 0.5 * f_al) * c + (0.5 + 0.5 * u) * g_al
        return (0.5 + 0.5 * o_al) * jnp.tanh(c), c

    z = jnp.zeros((B4, N, LW), jnp.float32)

    if packed:
        HB = LW // 2
        zq = jnp.zeros((HB, HB), jnp.float32)
        whh_pk = jnp.concatenate(
            [jnp.concatenate([whh_ref[0:HB, 0:HB], zq], axis=1),
             jnp.concatenate([zq, whh_ref[LW:LW + HB, LW:LW + HB]], axis=1)],
            axis=0)

        def step(i, carry):
            h, c = carry
            rf = i * N
            rb = (L - 1 - i) * N
            if not isinstance(i, int):
                rf = pl.multiple_of(rf, 8)
                rb = pl.multiple_of(rb, 8)
            g = (pre2[:, pl.ds(rf, N), 0:LW]
                 + pre2[:, pl.ds(rb, N), LW:2 * LW]
                 + _dot_rec(h, whh_pk))
            h, c = cell(g, c)
            for b in range(B4):
                hid_sm[pl.ds(rf, N), LW * b:LW * (b + 1)] = h[b]
                hid_sm[pl.ds(rb, N), LW * (B4 + b):LW * (B4 + b + 1)] = h[b]
            return h, c

        carry = (z, z)
    else:
        whhF = whh_ref[0:LW, 0:LW]
        whhB = whh_ref[LW:2 * LW, LW:2 * LW]

        def step(i, carry):
            hf, cf, hb, cb = carry
            rf = i * N
            rb = (L - 1 - i) * N
            if not isinstance(i, int):
                rf = pl.multiple_of(rf, 8)
                rb = pl.multiple_of(rb, 8)
            gf = pre2[:, pl.ds(rf, N), 0:LW] + _dot_rec(hf, whhF)
            gb = pre2[:, pl.ds(rb, N), LW:2 * LW] + _dot_rec(hb, whhB)
            hf, cf = cell(gf, cf)
            hb, cb = cell(gb, cb)
            for b in range(B4):
                hid_sm[pl.ds(rf, N), LW * b:LW * (b + 1)] = hf[b]
                hid_sm[pl.ds(rb, N), LW * (B4 + b):LW * (B4 + b + 1)] = hb[b]
            return hf, cf, hb, cb

        carry = (z, z, z, z)

    if N % 8 == 0:
        carry = lax.fori_loop(0, L, step, carry, unroll=4)
    else:
        for i in range(L):
            carry = step(i, carry)

    wlF = wl_ref[0:LW, :]
    if packed:
        wlB = jnp.concatenate(
            [jnp.zeros((LW // 2, C), jnp.float32),
             wl_ref[LW:LW + LW // 2, :]], axis=0)
    else:
        wlB = wl_ref[LW:2 * LW, :]
    wbigF = jnp.concatenate(
        [_dot(wlF, cstk[b]) for b in range(B4)], axis=0)
    wbigB = jnp.concatenate(
        [_dot(wlB, cstk[b]) for b in range(B4)], axis=0)
    lrep = jnp.concatenate([lb_ref[...]] * B4, axis=1)
    return (_dot_epi(hid_sm[0:R, 0:B4 * LW], wbigF)
            + _dot_epi(hid_sm[0:R, B4 * LW:2 * B4 * LW], wbigB)
            + lrep + X)


def _layer(a_ref, b_ref, perm_ref, q1_ref, q2_ref, p0, p1,
           cstk_ref, msum_ref, er_ref, ei_ref, scr,
           *, R_in, C_in, R_out, F, H, inverse, B4):
    T_in = R_in // F
    W = B4 * C_in
    y0 = _branch(a_ref, p0, cstk_ref, msum_ref, scr,
                 R=R_in, C=C_in, L=F, N=T_in, H=H, B4=B4)
    b_ref[0:R_in, 0:W] = _dot_perm(perm_ref[...], y0)
    y1 = _branch(b_ref, p1, cstk_ref, msum_ref, scr,
                 R=R_in, C=C_in, L=T_in, N=F, H=H, B4=B4)
    if not inverse:
        z1 = _dot_q(q1_ref[...], y1)
        z2 = _dot_q(q2_ref[...], y1)
        return _dot(z1, er_ref[...]) + _dot(z2, ei_ref[...])
    yr = _dot(y1, er_ref[...])
    yi = _dot(y1, ei_ref[...])
    return _dot_q(q1_ref[...], yr) + _dot_q(q2_ref[...], yi)


def _net_kernel(*args, B4, cfgs):
    x_ref = args[0]
    params = args[1:1 + 17 * 4]
    (cf_ref, ci_ref, mf_ref, mi_ref, er_ref, ei_ref,
     ert_ref, eit_ref, cft_ref) = args[69:78]
    o_ref = args[78]
    a_s, b_s, pre2, hid_sm = args[79:]
    scr = (pre2, hid_sm)

    xin = x_ref[...]
    cf = cf_ref[...]
    C0, R0 = xin.shape[1], xin.shape[2]
    acc = _dot(jnp.transpose(xin[0]), cf[0])
    for b in range(1, B4):
        acc = acc + _dot(jnp.transpose(xin[b]), cf[b])
    a_s[0:R0, 0:B4 * C0] = acc

    for li, cfg in enumerate(cfgs):
        p = params[li * 17:(li + 1) * 17]
        fwd = not cfg['inverse']
        out_val = _layer(
            a_s, b_s, p[0], p[1], p[2], p[3:10], p[10:17],
            cf_ref if fwd else ci_ref, mf_ref if fwd else mi_ref,
            ert_ref if fwd else er_ref, eit_ref if fwd else ei_ref,
            scr, B4=B4, **cfg)
        if li == 3:
            cft = cft_ref[...]
            for b in range(B4):
                o_ref[b, 0:C0, 0:cfg['R_out']] = jnp.transpose(
                    _dot(out_val, cft[b]))
        else:
            Wn = out_val.shape[1]
            a_s[0:cfg['R_out'], 0:Wn] = out_val


def kernel(x, l0_perm, l0_q1, l0_q2, l0_e1, l0_e2, l0_g0, l0_be0, l0_wih0, l0_bi0, l0_whh0, l0_wl0, l0_lb0, l0_g1, l0_be1, l0_wih1, l0_bi1, l0_whh1, l0_wl1, l0_lb1, l1_perm, l1_q1, l1_q2, l1_e1, l1_e2, l1_g0, l1_be0, l1_wih0, l1_bi0, l1_whh0, l1_wl0, l1_lb0, l1_g1, l1_be1, l1_wih1, l1_bi1, l1_whh1, l1_wl1, l1_lb1, l2_perm, l2_q1, l2_q2, l2_e1, l2_e2, l2_g0, l2_be0, l2_wih0, l2_bi0, l2_whh0, l2_wl0, l2_lb0, l2_g1, l2_be1, l2_wih1, l2_bi1, l2_whh1, l2_wl1, l2_lb1, l3_perm, l3_q1, l3_q2, l3_e1, l3_e2, l3_g0, l3_be0, l3_wih0, l3_bi0, l3_whh0, l3_wl0, l3_lb0, l3_g1, l3_be1, l3_wih1, l3_bi1, l3_whh1, l3_wl1, l3_lb1):
    B, C, F, T = x.shape
    a = x.reshape(B, C, F * T)

    per_layer = [
        [l0_perm, l0_q1, l0_q2, l0_g0, l0_be0, l0_wih0, l0_bi0, l0_whh0,
         l0_wl0, l0_lb0, l0_g1, l0_be1, l0_wih1, l0_bi1, l0_whh1, l0_wl1, l0_lb1],
        [l1_perm, l1_q1, l1_q2, l1_g0, l1_be0, l1_wih0, l1_bi0, l1_whh0,
         l1_wl0, l1_lb0, l1_g1, l1_be1, l1_wih1, l1_bi1, l1_whh1, l1_wl1, l1_lb1],
        [l2_perm, l2_q1, l2_q2, l2_g0, l2_be0, l2_wih0, l2_bi0, l2_whh0,
         l2_wl0, l2_lb0, l2_g1, l2_be1, l2_wih1, l2_bi1, l2_whh1, l2_wl1, l2_lb1],
        [l3_perm, l3_q1, l3_q2, l3_g0, l3_be0, l3_wih0, l3_bi0, l3_whh0,
         l3_wl0, l3_lb0, l3_g1, l3_be1, l3_wih1, l3_bi1, l3_whh1, l3_wl1, l3_lb1],
    ]
    consts = [arr for lp in per_layer for arr in lp]

    B4 = B // 2
    K = T // 2 + 1
    Rf, Ri = F * T, F * K
    C2 = 2 * C
    Wf, Wi = B4 * C, B4 * C2

    cf = np.zeros((B4, C, Wf), np.float32)
    for b in range(B4):
        cf[b, np.arange(C), b * C + np.arange(C)] = 1.0
    ci = np.zeros((B4, C2, Wi), np.float32)
    for b in range(B4):
        ci[b, np.arange(C2), b * C2 + np.arange(C2)] = 1.0
    mf = (np.add.outer(np.arange(Wf) // C, -(np.arange(Wf) // C)) == 0
          ).astype(np.float32)
    mi = (np.add.outer(np.arange(Wi) // C2, -(np.arange(Wi) // C2)) == 0
          ).astype(np.float32)
    er = np.zeros((Wi, Wf), np.float32)
    ei = np.zeros((Wi, Wf), np.float32)
    for j in range(Wf):
        er[C2 * (j // C) + (j % C), j] = 1.0
        ei[C2 * (j // C) + C + (j % C), j] = 1.0
    layout = [jnp.asarray(v) for v in
              (cf, ci, mf, mi, er, ei,
               er.T.copy(), ei.T.copy(), cf.transpose(0, 2, 1).copy())]

    cfgs = (
        dict(R_in=Rf, C_in=C, R_out=Ri, F=F, H=2 * C, inverse=False),
        dict(R_in=Ri, C_in=C2, R_out=Rf, F=F, H=4 * C, inverse=True),
        dict(R_in=Rf, C_in=C, R_out=Ri, F=F, H=2 * C, inverse=False),
        dict(R_in=Ri, C_in=C2, R_out=Rf, F=F, H=4 * C, inverse=True),
    )

    body = functools.partial(_net_kernel, B4=B4, cfgs=cfgs)
    out = pl.pallas_call(
        body,
        grid=(2,),
        in_specs=([pl.BlockSpec((B4, C, Rf), lambda i: (i, 0, 0))]
                  + [pl.BlockSpec(c.shape, functools.partial(
                        lambda n, i: (0,) * n, c.ndim))
                     for c in consts + layout]),
        out_specs=pl.BlockSpec((B4, C, Rf), lambda i: (i, 0, 0)),
        out_shape=jax.ShapeDtypeStruct((B, C, Rf), jnp.float32),
        scratch_shapes=[
            pltpu.VMEM((Rf, Wi), jnp.float32),
            pltpu.VMEM((Rf, Wi), jnp.float32),
            pltpu.VMEM((B4, Rf, 2 * LW), jnp.float32),
            pltpu.VMEM((Rf, 2 * B4 * LW), jnp.float32),
        ],
        compiler_params=pltpu.CompilerParams(
            dimension_semantics=("parallel",)),
    )(a, *consts, *layout)
    return out.reshape(B, C, F, T)


# speedup vs baseline: 1.0776x; 1.0010x over previous
import functools

import numpy as np

import jax
import jax.numpy as jnp
from jax import lax
from jax.experimental import pallas as pl
from jax.experimental.pallas import tpu as pltpu

LW = 128


def _dot(a, w):
    return lax.dot_general(a, w, (((a.ndim - 1,), (0,)), ((), ())),
                           preferred_element_type=jnp.float32)


def _dot_rec(a, w):
    return lax.dot_general(a, w, (((a.ndim - 1,), (0,)), ((), ())),
                           preferred_element_type=jnp.float32)


def _dot_pre(a, w):
    return lax.dot_general(a, w, (((a.ndim - 1,), (0,)), ((), ())),
                           preferred_element_type=jnp.float32)


def _dot_perm(a, w):
    return lax.dot_general(a, w, (((a.ndim - 1,), (0,)), ((), ())),
                           preferred_element_type=jnp.float32)


def _dot_q(a, w):
    return lax.dot_general(a, w, (((a.ndim - 1,), (0,)), ((), ())),
                           preferred_element_type=jnp.float32)


def _dot_epi(a, w):
    return lax.dot_general(a, w, (((a.ndim - 1,), (0,)), ((), ())),
                           preferred_element_type=jnp.float32)


def _branch(x_ref, pars, cstk_ref, msum_ref, scr, *, R, C, L, N, H, B4):
    g_ref, be_ref, wih_ref, bi_ref, whh_ref, wl_ref, lb_ref = pars
    pre2, hid_sm = scr
    W = B4 * C
    packed = 8 * H <= LW
    X = x_ref[0:R, 0:W]

    inv_n = 1.0 / (R * C)
    msum = msum_ref[...]
    s1 = jnp.sum(X, axis=0, keepdims=True)
    mu = _dot(s1, msum) * inv_n
    d = X - mu
    s2 = jnp.sum(d * d, axis=0, keepdims=True)
    var = _dot(s2, msum) * inv_n
    rs = lax.rsqrt(var + 1e-5)
    gam, bet = g_ref[...], be_ref[...]
    grep = jnp.concatenate([gam] * B4, axis=1)
    brep = jnp.concatenate([bet] * B4, axis=1)
    xn = d * (rs * grep) + brep

    cstk = cstk_ref[...]

    if packed:
        G4 = 4 * H
        zc = jnp.zeros((C, LW - G4), jnp.float32)
        zb = jnp.zeros((1, LW - G4), jnp.float32)
        dirs = (
            (0, jnp.concatenate([wih_ref[:, 0:G4], zc], axis=1),
             jnp.concatenate([bi_ref[:, 0:G4], zb], axis=1)),
            (LW, jnp.concatenate([zc, wih_ref[:, LW:LW + G4]], axis=1),
             jnp.concatenate([zb, bi_ref[:, LW:LW + G4]], axis=1)),
        )
    else:
        dirs = (
            (0, wih_ref[:, 0:LW], bi_ref[:, 0:LW]),
            (LW, wih_ref[:, LW:2 * LW], bi_ref[:, LW:2 * LW]),
        )

    for lane0, wihD, biD in dirs:
        wih_ext = jnp.concatenate(
            [lax.dot_general(cstk[b], wihD, (((0,), (0,)), ((), ())),
                             preferred_element_type=jnp.float32)
             for b in range(B4)], axis=1)
        PD = _dot_pre(xn, wih_ext)
        for b in range(B4):
            pre2[b, 0:R, lane0:lane0 + LW] = (
                PD[:, LW * b:LW * (b + 1)] + biD)

    lane = lax.broadcasted_iota(jnp.int32, (1, 1, LW), 2) % (4 * H)
    mhalf = jnp.where((lane >= 2 * H) & (lane < 3 * H), 1.0, 0.5)

    def cell(gts, c):
        u = jnp.tanh(gts * mhalf)
        f_al = pltpu.roll(u, LW - H, axis=2)
        g_al = pltpu.roll(u, LW - 2 * H, axis=2)
        o_al = pltpu.roll(u, LW - 3 * H, axis=2)
        c = (0.5 + 0.5 * f_al) * c + (0.5 + 0.5 * u) * g_al
        return (0.5 + 0.5 * o_al) * jnp.tanh(c), c

    z = jnp.zeros((B4, N, LW), jnp.float32)

    if packed:
        HB = LW // 2
        zq = jnp.zeros((HB, HB), jnp.float32)
        whh_pk = jnp.concatenate(
            [jnp.concatenate([whh_ref[0:HB, 0:HB], zq], axis=1),
             jnp.concatenate([zq, whh_ref[LW:LW + HB, LW:LW + HB]], axis=1)],
            axis=0)

        def step(i, carry):
            h, c = carry
            rf = i * N
            rb = (L - 1 - i) * N
            if not isinstance(i, int):
                rf = pl.multiple_of(rf, 8)
                rb = pl.multiple_of(rb, 8)
            g = (pre2[:, pl.ds(rf, N), 0:LW]
                 + pre2[:, pl.ds(rb, N), LW:2 * LW]
                 + _dot_rec(h, whh_pk))
            h, c = cell(g, c)
            for b in range(B4):
                hid_sm[pl.ds(rf, N), LW * b:LW * (b + 1)] = h[b]
                hid_sm[pl.ds(rb, N), LW * (B4 + b):LW * (B4 + b + 1)] = h[b]
            return h, c

        carry = (z, z)
    else:
        whhF = whh_ref[0:LW, 0:LW]
        whhB = whh_ref[LW:2 * LW, LW:2 * LW]

        def step(i, carry):
            hf, cf, hb, cb = carry
            rf = i * N
            rb = (L - 1 - i) * N
            if not isinstance(i, int):
                rf = pl.multiple_of(rf, 8)
                rb = pl.multiple_of(rb, 8)
            gf = pre2[:, pl.ds(rf, N), 0:LW] + _dot_rec(hf, whhF)
            gb = pre2[:, pl.ds(rb, N), LW:2 * LW] + _dot_rec(hb, whhB)
            hf, cf = cell(gf, cf)
            hb, cb = cell(gb, cb)
            for b in range(B4):
                hid_sm[pl.ds(rf, N), LW * b:LW * (b + 1)] = hf[b]
                hid_sm[pl.ds(rb, N), LW * (B4 + b):LW * (B4 + b + 1)] = hb[b]
            return hf, cf, hb, cb

        carry = (z, z, z, z)

    if N % 8 == 0:
        carry = lax.fori_loop(0, L, step, carry, unroll=4)
    else:
        for i in range(L):
            carry = step(i, carry)

    wlF = wl_ref[0:LW, :]
    if packed:
        wlB = jnp.concatenate(
            [jnp.zeros((LW // 2, C), jnp.float32),
             wl_ref[LW:LW + LW // 2, :]], axis=0)
    else:
        wlB = wl_ref[LW:2 * LW, :]
    wbigF = jnp.concatenate(
        [_dot(wlF, cstk[b]) for b in range(B4)], axis=0)
    wbigB = jnp.concatenate(
        [_dot(wlB, cstk[b]) for b in range(B4)], axis=0)
    lrep = jnp.concatenate([lb_ref[...]] * B4, axis=1)
    return (_dot_epi(hid_sm[0:R, 0:B4 * LW], wbigF)
            + _dot_epi(hid_sm[0:R, B4 * LW:2 * B4 * LW], wbigB)
            + lrep + X)


def _layer(a_ref, b_ref, perm_ref, q1_ref, q2_ref, p0, p1,
           cstk_ref, msum_ref, er_ref, ei_ref, scr,
           *, R_in, C_in, R_out, F, H, inverse, B4):
    T_in = R_in // F
    W = B4 * C_in
    y0 = _branch(a_ref, p0, cstk_ref, msum_ref, scr,
                 R=R_in, C=C_in, L=F, N=T_in, H=H, B4=B4)
    b_ref[0:R_in, 0:W] = _dot_perm(perm_ref[...], y0)
    y1 = _branch(b_ref, p1, cstk_ref, msum_ref, scr,
                 R=R_in, C=C_in, L=T_in, N=F, H=H, B4=B4)
    if not inverse:
        z1 = _dot_q(q1_ref[...], y1)
        z2 = _dot_q(q2_ref[...], y1)
        return _dot(z1, er_ref[...]) + _dot(z2, ei_ref[...])
    yr = _dot(y1, er_ref[...])
    yi = _dot(y1, ei_ref[...])
    return _dot_q(q1_ref[...], yr) + _dot_q(q2_ref[...], yi)


def _net_kernel(*args, B4, cfgs):
    x_ref = args[0]
    params = args[1:1 + 17 * 4]
    (cf_ref, ci_ref, mf_ref, mi_ref, er_ref, ei_ref,
     ert_ref, eit_ref, cft_ref) = args[69:78]
    o_ref = args[78]
    a_s, b_s, pre2, hid_sm = args[79:]
    scr = (pre2, hid_sm)

    xin = x_ref[...]
    cf = cf_ref[...]
    C0, R0 = xin.shape[1], xin.shape[2]
    acc = _dot(jnp.transpose(xin[0]), cf[0])
    for b in range(1, B4):
        acc = acc + _dot(jnp.transpose(xin[b]), cf[b])
    a_s[0:R0, 0:B4 * C0] = acc

    for li, cfg in enumerate(cfgs):
        p = params[li * 17:(li + 1) * 17]
        fwd = not cfg['inverse']
        out_val = _layer(
            a_s, b_s, p[0], p[1], p[2], p[3:10], p[10:17],
            cf_ref if fwd else ci_ref, mf_ref if fwd else mi_ref,
            ert_ref if fwd else er_ref, eit_ref if fwd else ei_ref,
            scr, B4=B4, **cfg)
        if li == 3:
            cft = cft_ref[...]
            for b in range(B4):
                o_ref[b, 0:C0, 0:cfg['R_out']] = jnp.transpose(
                    _dot(out_val, cft[b]))
        else:
            Wn = out_val.shape[1]
            a_s[0:cfg['R_out'], 0:Wn] = out_val


def kernel(x, l0_perm, l0_q1, l0_q2, l0_e1, l0_e2, l0_g0, l0_be0, l0_wih0, l0_bi0, l0_whh0, l0_wl0, l0_lb0, l0_g1, l0_be1, l0_wih1, l0_bi1, l0_whh1, l0_wl1, l0_lb1, l1_perm, l1_q1, l1_q2, l1_e1, l1_e2, l1_g0, l1_be0, l1_wih0, l1_bi0, l1_whh0, l1_wl0, l1_lb0, l1_g1, l1_be1, l1_wih1, l1_bi1, l1_whh1, l1_wl1, l1_lb1, l2_perm, l2_q1, l2_q2, l2_e1, l2_e2, l2_g0, l2_be0, l2_wih0, l2_bi0, l2_whh0, l2_wl0, l2_lb0, l2_g1, l2_be1, l2_wih1, l2_bi1, l2_whh1, l2_wl1, l2_lb1, l3_perm, l3_q1, l3_q2, l3_e1, l3_e2, l3_g0, l3_be0, l3_wih0, l3_bi0, l3_whh0, l3_wl0, l3_lb0, l3_g1, l3_be1, l3_wih1, l3_bi1, l3_whh1, l3_wl1, l3_lb1):
    B, C, F, T = x.shape
    a = x.reshape(B, C, F * T)

    per_layer = [
        [l0_perm, l0_q1, l0_q2, l0_g0, l0_be0, l0_wih0, l0_bi0, l0_whh0,
         l0_wl0, l0_lb0, l0_g1, l0_be1, l0_wih1, l0_bi1, l0_whh1, l0_wl1, l0_lb1],
        [l1_perm, l1_q1, l1_q2, l1_g0, l1_be0, l1_wih0, l1_bi0, l1_whh0,
         l1_wl0, l1_lb0, l1_g1, l1_be1, l1_wih1, l1_bi1, l1_whh1, l1_wl1, l1_lb1],
        [l2_perm, l2_q1, l2_q2, l2_g0, l2_be0, l2_wih0, l2_bi0, l2_whh0,
         l2_wl0, l2_lb0, l2_g1, l2_be1, l2_wih1, l2_bi1, l2_whh1, l2_wl1, l2_lb1],
        [l3_perm, l3_q1, l3_q2, l3_g0, l3_be0, l3_wih0, l3_bi0, l3_whh0,
         l3_wl0, l3_lb0, l3_g1, l3_be1, l3_wih1, l3_bi1, l3_whh1, l3_wl1, l3_lb1],
    ]
    consts = [arr for lp in per_layer for arr in lp]

    B4 = B // 2
    K = T // 2 + 1
    Rf, Ri = F * T, F * K
    C2 = 2 * C
    Wf, Wi = B4 * C, B4 * C2

    cf = np.zeros((B4, C, Wf), np.float32)
    for b in range(B4):
        cf[b, np.arange(C), b * C + np.arange(C)] = 1.0
    ci = np.zeros((B4, C2, Wi), np.float32)
    for b in range(B4):
        ci[b, np.arange(C2), b * C2 + np.arange(C2)] = 1.0
    mf = (np.add.outer(np.arange(Wf) // C, -(np.arange(Wf) // C)) == 0
          ).astype(np.float32)
    mi = (np.add.outer(np.arange(Wi) // C2, -(np.arange(Wi) // C2)) == 0
          ).astype(np.float32)
    er = np.zeros((Wi, Wf), np.float32)
    ei = np.zeros((Wi, Wf), np.float32)
    for j in range(Wf):
        er[C2 * (j // C) + (j % C), j] = 1.0
        ei[C2 * (j // C) + C + (j % C), j] = 1.0
    layout = [jnp.asarray(v) for v in
              (cf, ci, mf, mi, er, ei,
               er.T.copy(), ei.T.copy(), cf.transpose(0, 2, 1).copy())]

    cfgs = (
        dict(R_in=Rf, C_in=C, R_out=Ri, F=F, H=2 * C, inverse=False),
        dict(R_in=Ri, C_in=C2, R_out=Rf, F=F, H=4 * C, inverse=True),
        dict(R_in=Rf, C_in=C, R_out=Ri, F=F, H=2 * C, inverse=False),
        dict(R_in=Ri, C_in=C2, R_out=Rf, F=F, H=4 * C, inverse=True),
    )

    body = functools.partial(_net_kernel, B4=B4, cfgs=cfgs)
    out = pl.pallas_call(
        body,
        grid=(2,),
        in_specs=([pl.BlockSpec((B4, C, Rf), lambda i: (i, 0, 0))]
                  + [pl.BlockSpec(c.shape, functools.partial(
                        lambda n, i: (0,) * n, c.ndim))
                     for c in consts + layout]),
        out_specs=pl.BlockSpec((B4, C, Rf), lambda i: (i, 0, 0)),
        out_shape=jax.ShapeDtypeStruct((B, C, Rf), jnp.float32),
        scratch_shapes=[
            pltpu.VMEM((Rf, Wi), jnp.float32),
            pltpu.VMEM((Rf, Wi), jnp.float32),
            pltpu.VMEM((B4, Rf, 2 * LW), jnp.float32),
            pltpu.VMEM((Rf, 2 * B4 * LW), jnp.float32),
        ],
        compiler_params=pltpu.CompilerParams(
            dimension_semantics=("parallel",)),
    )(a, *consts, *layout)
    return out.reshape(B, C, F, T)
